# baseline (speedup 1.0000x reference)
import functools

import jax
import jax.numpy as jnp
from jax import lax
from jax.experimental import pallas as pl
from jax.experimental.pallas import tpu as pltpu
from jax.experimental.pallas import tpu_sc as plsc

N = 10000
NP = 10240
H = 32
K = 15
KP_EXTENT = 0.6
NEG = 0.1
R = 128
RH = R * H

NW = 32
CH = 512
CHD = 1024
M = NP * H


def _lrelu(x):
    return jnp.where(x >= 0, x, NEG * x)


CHB = 256
NB = 3


def _sc_gather(table, idx2d):
    m = idx2d.shape[0] * 128
    per_w = m // NW
    n_ch = per_w // CHB
    n_ir = per_w // 128
    mesh = plsc.VectorSubcoreMesh(core_axis_name="c", subcore_axis_name="s")

    @functools.partial(
        pl.kernel,
        mesh=mesh,
        out_type=jax.ShapeDtypeStruct((m, 128), jnp.float32),
        scratch_types=[
            pltpu.VMEM((n_ir, 128), jnp.int32),
            pltpu.VMEM((NB * CHB, 128), jnp.float32),
        ] + [pltpu.SemaphoreType.DMA] * (2 * NB),
    )
    def k(table_hbm, idx_hbm, out_hbm, idx_v, rows_v, *sems):
        gsems, wsems = sems[:NB], sems[NB:]
        wid = lax.axis_index("s") * 2 + lax.axis_index("c")
        base = pl.multiple_of(wid * per_w, per_w)
        row0 = pl.multiple_of(base // 128, 8)
        pltpu.sync_copy(idx_hbm.at[pl.ds(row0, n_ir)], idx_v)

        def gather(ch, b):
            return [
                pltpu.async_copy(
                    table_hbm.at[idx_v.at[ch * (CHB // 128) + j]],
                    rows_v.at[pl.ds(b * CHB + j * 128, 128)], gsems[b])
                for j in range(CHB // 128)
            ]

        writes = {}
        pending = gather(0, 0)
        for ch in range(n_ch):
            b = ch % NB
            nxt = None
            if ch + 1 < n_ch:
                nb = (ch + 1) % NB
                if nb in writes:
                    writes[nb].wait()
                nxt = gather(ch + 1, nb)
            for g in pending:
                g.wait()
            writes[b] = pltpu.async_copy(
                rows_v.at[pl.ds(b * CHB, CHB)],
                out_hbm.at[pl.ds(base + ch * CHB, CHB)], wsems[b])
            pending = nxt
        for b in writes:
            writes[b].wait()

    return k(table, idx2d)


def _sc_dnb(px, py, pz, idx2d):
    per_w = M // NW
    n_out = per_w // CHD
    mesh = plsc.VectorSubcoreMesh(core_axis_name="c", subcore_axis_name="s")

    @functools.partial(
        pl.kernel,
        mesh=mesh,
        out_type=jax.ShapeDtypeStruct((8, M), jnp.float32),
        scratch_types=[
            pltpu.VMEM((NP,), jnp.float32),
            pltpu.VMEM((NP,), jnp.float32),
            pltpu.VMEM((NP,), jnp.float32),
            pltpu.VMEM((8, 128), jnp.int32),
            pltpu.VMEM((8, CHD), jnp.float32),
        ],
        compiler_params=pltpu.CompilerParams(needs_layout_passes=False),
    )
    def k(px_hbm, py_hbm, pz_hbm, idx_hbm, out_hbm, px_v, py_v, pz_v,
          idx_v, rows_v):
        wid = lax.axis_index("s") * 2 + lax.axis_index("c")
        base = wid * per_w
        pltpu.sync_copy(px_hbm, px_v)
        pltpu.sync_copy(py_hbm, py_v)
        pltpu.sync_copy(pz_hbm, pz_v)

        def body(c, _):
            off = pl.multiple_of(base + c * CHD, CHD)
            row0 = pl.multiple_of(off // 128, 8)
            pltpu.sync_copy(idx_hbm.at[pl.ds(row0, 8)], idx_v)
            for j in range(8):
                for t in range(8):
                    col = j * 128 + t * 16
                    iv = idx_v[j, t * 16:(t + 1) * 16]
                    qi = (off + col) // 32
                    qs = jnp.full((16,), qi, jnp.int32)
                    dx = plsc.load_gather(px_v, [iv]) - \
                        plsc.load_gather(px_v, [qs])
                    dy = plsc.load_gather(py_v, [iv]) - \
                        plsc.load_gather(py_v, [qs])
                    dz = plsc.load_gather(pz_v, [iv]) - \
                        plsc.load_gather(pz_v, [qs])
                    rows_v[0, col:col + 16] = dx
                    rows_v[1, col:col + 16] = dy
                    rows_v[2, col:col + 16] = dz
                    rows_v[3, col:col + 16] = dx * dx + dy * dy + dz * dz
            pltpu.sync_copy(rows_v, out_hbm.at[:, pl.ds(off, CHD)])
            return 0

        lax.fori_loop(0, n_out, body, 0)

    return k(px, py, pz, idx2d)


def _influence_t(dnb, krt):
    dxt, dyt, dzt, nbt = dnb[0:1, :], dnb[1:2, :], dnb[2:3, :], dnb[3:4, :]
    kx, ky, kz = krt[:, 0:1], krt[:, 1:2], krt[:, 2:3]
    ksq = krt[:, 3:4]
    sqt = nbt + ksq - 2.0 * (kx * dxt + ky * dyt + kz * dzt)
    rt = jnp.sqrt(jnp.maximum(sqt, 1e-12))
    inflt = jnp.maximum(1.0 - rt / KP_EXTENT, 0.0)
    row = lax.broadcasted_iota(jnp.int32, (16, RH), 0)
    return jnp.where(row < K, inflt, 0.0)


def _kpconv_block(nx, dnb, krt, w_ref, wx_ref, ddim):
    inflt = _influence_t(dnb, krt)
    sel = jnp.where(
        lax.broadcasted_iota(jnp.int32, (128, 16), 0) // 8
        == lax.broadcasted_iota(jnp.int32, (128, 16), 1), 1.0, 0.0)
    bmask = jnp.where(
        lax.broadcasted_iota(jnp.int32, (128, 256), 0) % 8
        == lax.broadcasted_iota(jnp.int32, (128, 256), 1) // 32, 1.0, 0.0)
    for s in range(16):
        inflt_s = inflt[:, s * 256:(s + 1) * 256]
        t2 = jnp.dot(sel, inflt_s,
                     preferred_element_type=jnp.float32) * bmask
        wx_s = jnp.dot(t2, nx[s * 256:(s + 1) * 256, :],
                       preferred_element_type=jnp.float32)
        for kk in range(K):
            wx_ref[pl.ds(kk * R + s * 8, 8), :] = wx_s[kk * 8:(kk + 1) * 8, :]
    acc = jnp.zeros((R, ddim), jnp.float32)
    for kk in range(K):
        acc = acc + jnp.dot(wx_ref[pl.ds(kk * R, R), :], w_ref[kk],
                            preferred_element_type=jnp.float32)
    return acc


def _tc_a_body(g_ref, dnb_ref, krt_ref, w1_ref, b1_ref, u1w_ref, u1b_ref,
               sw_ref, sb_ref, y_ref, sc_ref, wx_ref):
    acc = _kpconv_block(g_ref[...], dnb_ref[...], krt_ref[...], w1_ref,
                        wx_ref, 256)
    x = _lrelu(acc + b1_ref[...])
    y = _lrelu(jnp.dot(x, u1w_ref[...],
                       preferred_element_type=jnp.float32) + u1b_ref[...])
    sc = jnp.dot(x, sw_ref[...],
                 preferred_element_type=jnp.float32) + sb_ref[...]
    y_ref[...] = y
    sc_ref[...] = sc


def _tc_b_body(g_ref, dnb_ref, krt_ref, w2_ref, b2_ref, u2w_ref, u2b_ref,
               bw_ref, bb_ref, sc_ref, out_ref, wx_ref):
    acc = _kpconv_block(g_ref[...], dnb_ref[...], krt_ref[...], w2_ref,
                        wx_ref, 128)
    y2 = _lrelu(acc + b2_ref[...])
    y3 = jnp.dot(y2, u2w_ref[...],
                 preferred_element_type=jnp.float32) + u2b_ref[...]
    xb = _lrelu(y3 + sc_ref[...])
    out_ref[...] = jnp.dot(xb, bw_ref[...],
                           preferred_element_type=jnp.float32) + bb_ref[...]


def _const_spec(shape):
    nd = len(shape)
    return pl.BlockSpec(shape, lambda i: (0,) * nd)


def _tc_a(g1, dnb, krt, w1, b1, u1w, u1b, sw, sb, blk0):
    nblk = g1.shape[0] // RH
    return pl.pallas_call(
        _tc_a_body,
        grid=(nblk,),
        in_specs=[
            pl.BlockSpec((RH, 128), lambda i: (i, 0)),
            pl.BlockSpec((8, RH), lambda i: (0, i + blk0)),
            _const_spec((16, 8)),
            _const_spec((K, 128, 256)),
            _const_spec((1, 256)),
            _const_spec((256, 128)),
            _const_spec((1, 128)),
            _const_spec((256, 512)),
            _const_spec((1, 512)),
        ],
        out_specs=[
            pl.BlockSpec((R, 128), lambda i: (i, 0)),
            pl.BlockSpec((R, 512), lambda i: (i, 0)),
        ],
        out_shape=[
            jax.ShapeDtypeStruct((nblk * R, 128), jnp.float32),
            jax.ShapeDtypeStruct((nblk * R, 512), jnp.float32),
        ],
        scratch_shapes=[pltpu.VMEM((K * R, 128), jnp.float32)],
    )(g1, dnb, krt, w1, b1, u1w, u1b, sw, sb)


def _tc_b(g2, dnb, krt, w2, b2, u2w, u2b, bw, bb, sc, blk0):
    nblk = g2.shape[0] // RH
    return pl.pallas_call(
        _tc_b_body,
        grid=(nblk,),
        in_specs=[
            pl.BlockSpec((RH, 128), lambda i: (i, 0)),
            pl.BlockSpec((8, RH), lambda i: (0, i + blk0)),
            _const_spec((16, 8)),
            _const_spec((K, 128, 128)),
            _const_spec((1, 128)),
            _const_spec((128, 512)),
            _const_spec((1, 512)),
            _const_spec((512, 256)),
            _const_spec((1, 256)),
            pl.BlockSpec((R, 512), lambda i: (i, 0)),
        ],
        out_specs=pl.BlockSpec((R, 256), lambda i: (i, 0)),
        out_shape=jax.ShapeDtypeStruct((nblk * R, 256), jnp.float32),
        scratch_shapes=[pltpu.VMEM((K * R, 128), jnp.float32)],
    )(g2, dnb, krt, w2, b2, u2w, u2b, bw, bb, sc)


def _krt(kp):
    base = jnp.concatenate([kp, jnp.sum(kp * kp, 1, keepdims=True)], 1)
    return jnp.pad(base, ((0, 1), (0, 4)))


def kernel(features, points, neighbors, kp1, w1, b1, u1w, u1b, kp2, w2, b2,
           u2w, u2b, sw, sb, bw, bb):
    idx2d = jnp.pad(neighbors, ((0, NP - N), (0, 0))).reshape(M // 128, 128)
    px = jnp.pad(points[:, 0], (0, NP - N))
    py = jnp.pad(points[:, 1], (0, NP - N))
    pz = jnp.pad(points[:, 2], (0, NP - N))
    kr1 = _krt(kp1)
    kr2 = _krt(kp2)
    b1r = b1.reshape(1, -1)
    u1br = u1b.reshape(1, -1)
    sbr = sb.reshape(1, -1)
    b2r = b2.reshape(1, -1)
    u2br = u2b.reshape(1, -1)
    bbr = bb.reshape(1, -1)

    S = 10
    srow = M // S // 128
    sblk = NP // R // S
    idx_s = [idx2d[i * srow:(i + 1) * srow] for i in range(S)]

    dnb = _sc_dnb(px, py, pz, idx2d)
    g1 = [_sc_gather(features, ix) for ix in idx_s]
    ta = [_tc_a(g1[i], dnb, kr1, w1, b1r, u1w, u1br, sw, sbr, i * sblk)
          for i in range(S)]
    y = jnp.concatenate([t[0] for t in ta], 0)
    g2 = [_sc_gather(y, ix) for ix in idx_s]
    outs = [_tc_b(g2[i], dnb, kr2, w2, b2r, u2w, u2br, bw, bbr, ta[i][1],
                  i * sblk) for i in range(S)]
    out = jnp.concatenate(outs, 0)
    return out[:N].T[None]


# speedup vs baseline: 1.0098x; 1.0098x over previous
import functools

import jax
import jax.numpy as jnp
from jax import lax
from jax.experimental import pallas as pl
from jax.experimental.pallas import tpu as pltpu
from jax.experimental.pallas import tpu_sc as plsc

N = 10000
NP = 10240
H = 32
K = 15
KP_EXTENT = 0.6
NEG = 0.1
R = 128
RH = R * H

NW = 32
CH = 512
CHD = 1024
M = NP * H


def _lrelu(x):
    return jnp.where(x >= 0, x, NEG * x)


CHB = 256
NB = 3


def _sc_gather(table, idx2d):
    m = idx2d.shape[0] * 128
    per_w = m // NW
    n_ch = per_w // CHB
    n_ir = per_w // 128
    mesh = plsc.VectorSubcoreMesh(core_axis_name="c", subcore_axis_name="s")

    @functools.partial(
        pl.kernel,
        mesh=mesh,
        out_type=jax.ShapeDtypeStruct((m, 128), jnp.float32),
        scratch_types=[
            pltpu.VMEM((n_ir, 128), jnp.int32),
            pltpu.VMEM((NB * CHB, 128), jnp.float32),
        ] + [pltpu.SemaphoreType.DMA] * (2 * NB),
    )
    def k(table_hbm, idx_hbm, out_hbm, idx_v, rows_v, *sems):
        gsems, wsems = sems[:NB], sems[NB:]
        wid = lax.axis_index("s") * 2 + lax.axis_index("c")
        base = pl.multiple_of(wid * per_w, per_w)
        row0 = pl.multiple_of(base // 128, 8)
        pltpu.sync_copy(idx_hbm.at[pl.ds(row0, n_ir)], idx_v)

        def gather(ch, b):
            return [
                pltpu.async_copy(
                    table_hbm.at[idx_v.at[ch * (CHB // 128) + j]],
                    rows_v.at[pl.ds(b * CHB + j * 128, 128)], gsems[b])
                for j in range(CHB // 128)
            ]

        writes = {}
        pending = gather(0, 0)
        for ch in range(n_ch):
            b = ch % NB
            nxt = None
            if ch + 1 < n_ch:
                nb = (ch + 1) % NB
                if nb in writes:
                    writes[nb].wait()
                nxt = gather(ch + 1, nb)
            for g in pending:
                g.wait()
            writes[b] = pltpu.async_copy(
                rows_v.at[pl.ds(b * CHB, CHB)],
                out_hbm.at[pl.ds(base + ch * CHB, CHB)], wsems[b])
            pending = nxt
        for b in writes:
            writes[b].wait()

    return k(table, idx2d)


def _sc_dnb(px, py, pz, idx2d):
    per_w = M // NW
    n_out = per_w // CHD
    mesh = plsc.VectorSubcoreMesh(core_axis_name="c", subcore_axis_name="s")

    @functools.partial(
        pl.kernel,
        mesh=mesh,
        out_type=jax.ShapeDtypeStruct((8, M), jnp.float32),
        scratch_types=[
            pltpu.VMEM((NP,), jnp.float32),
            pltpu.VMEM((NP,), jnp.float32),
            pltpu.VMEM((NP,), jnp.float32),
            pltpu.VMEM((8, 128), jnp.int32),
            pltpu.VMEM((8, CHD), jnp.float32),
        ],
        compiler_params=pltpu.CompilerParams(needs_layout_passes=False),
    )
    def k(px_hbm, py_hbm, pz_hbm, idx_hbm, out_hbm, px_v, py_v, pz_v,
          idx_v, rows_v):
        wid = lax.axis_index("s") * 2 + lax.axis_index("c")
        base = wid * per_w
        pltpu.sync_copy(px_hbm, px_v)
        pltpu.sync_copy(py_hbm, py_v)
        pltpu.sync_copy(pz_hbm, pz_v)

        def body(c, _):
            off = pl.multiple_of(base + c * CHD, CHD)
            row0 = pl.multiple_of(off // 128, 8)
            pltpu.sync_copy(idx_hbm.at[pl.ds(row0, 8)], idx_v)
            for j in range(8):
                for t in range(8):
                    col = j * 128 + t * 16
                    iv = idx_v[j, t * 16:(t + 1) * 16]
                    qi = (off + col) // 32
                    qs = jnp.full((16,), qi, jnp.int32)
                    dx = plsc.load_gather(px_v, [iv]) - \
                        plsc.load_gather(px_v, [qs])
                    dy = plsc.load_gather(py_v, [iv]) - \
                        plsc.load_gather(py_v, [qs])
                    dz = plsc.load_gather(pz_v, [iv]) - \
                        plsc.load_gather(pz_v, [qs])
                    rows_v[0, col:col + 16] = dx
                    rows_v[1, col:col + 16] = dy
                    rows_v[2, col:col + 16] = dz
                    rows_v[3, col:col + 16] = dx * dx + dy * dy + dz * dz
            pltpu.sync_copy(rows_v, out_hbm.at[:, pl.ds(off, CHD)])
            return 0

        lax.fori_loop(0, n_out, body, 0)

    return k(px, py, pz, idx2d)


def _influence_t(dnb, krt):
    dxt, dyt, dzt, nbt = dnb[0:1, :], dnb[1:2, :], dnb[2:3, :], dnb[3:4, :]
    kx, ky, kz = krt[:, 0:1], krt[:, 1:2], krt[:, 2:3]
    ksq = krt[:, 3:4]
    sqt = nbt + ksq - 2.0 * (kx * dxt + ky * dyt + kz * dzt)
    rt = jnp.sqrt(jnp.maximum(sqt, 1e-12))
    inflt = jnp.maximum(1.0 - rt / KP_EXTENT, 0.0)
    row = lax.broadcasted_iota(jnp.int32, (16, RH), 0)
    return jnp.where(row < K, inflt, 0.0)


def _kpconv_block(nx, dnb, krt, w_ref, wx_ref, ddim):
    inflt = _influence_t(dnb, krt)
    sel = jnp.where(
        lax.broadcasted_iota(jnp.int32, (128, 16), 0) // 8
        == lax.broadcasted_iota(jnp.int32, (128, 16), 1), 1.0, 0.0)
    bmask = jnp.where(
        lax.broadcasted_iota(jnp.int32, (128, 256), 0) % 8
        == lax.broadcasted_iota(jnp.int32, (128, 256), 1) // 32, 1.0, 0.0)
    for s in range(16):
        inflt_s = inflt[:, s * 256:(s + 1) * 256]
        t2 = jnp.dot(sel, inflt_s,
                     preferred_element_type=jnp.float32) * bmask
        wx_s = jnp.dot(t2, nx[s * 256:(s + 1) * 256, :],
                       preferred_element_type=jnp.float32)
        for kk in range(K):
            wx_ref[pl.ds(kk * R + s * 8, 8), :] = wx_s[kk * 8:(kk + 1) * 8, :]
    acc = jnp.zeros((R, ddim), jnp.float32)
    for kk in range(K):
        acc = acc + jnp.dot(wx_ref[pl.ds(kk * R, R), :], w_ref[kk],
                            preferred_element_type=jnp.float32)
    return acc


def _tc_a_body(g_ref, dnb_ref, krt_ref, w1_ref, b1_ref, u1w_ref, u1b_ref,
               sw_ref, sb_ref, y_ref, sc_ref, wx_ref):
    acc = _kpconv_block(g_ref[...], dnb_ref[...], krt_ref[...], w1_ref,
                        wx_ref, 256)
    x = _lrelu(acc + b1_ref[...])
    y = _lrelu(jnp.dot(x, u1w_ref[...],
                       preferred_element_type=jnp.float32) + u1b_ref[...])
    sc = jnp.dot(x, sw_ref[...],
                 preferred_element_type=jnp.float32) + sb_ref[...]
    y_ref[...] = y
    sc_ref[...] = sc


def _tc_b_body(g_ref, dnb_ref, krt_ref, w2_ref, b2_ref, u2w_ref, u2b_ref,
               bw_ref, bb_ref, sc_ref, out_ref, wx_ref):
    acc = _kpconv_block(g_ref[...], dnb_ref[...], krt_ref[...], w2_ref,
                        wx_ref, 128)
    y2 = _lrelu(acc + b2_ref[...])
    y3 = jnp.dot(y2, u2w_ref[...],
                 preferred_element_type=jnp.float32) + u2b_ref[...]
    xb = _lrelu(y3 + sc_ref[...])
    out_ref[...] = jnp.dot(xb, bw_ref[...],
                           preferred_element_type=jnp.float32) + bb_ref[...]


def _const_spec(shape):
    nd = len(shape)
    return pl.BlockSpec(shape, lambda i: (0,) * nd)


def _tc_a(g1, dnb, krt, w1, b1, u1w, u1b, sw, sb, blk0):
    nblk = g1.shape[0] // RH
    return pl.pallas_call(
        _tc_a_body,
        grid=(nblk,),
        in_specs=[
            pl.BlockSpec((RH, 128), lambda i: (i, 0)),
            pl.BlockSpec((8, RH), lambda i: (0, i + blk0)),
            _const_spec((16, 8)),
            _const_spec((K, 128, 256)),
            _const_spec((1, 256)),
            _const_spec((256, 128)),
            _const_spec((1, 128)),
            _const_spec((256, 512)),
            _const_spec((1, 512)),
        ],
        out_specs=[
            pl.BlockSpec((R, 128), lambda i: (i, 0)),
            pl.BlockSpec((R, 512), lambda i: (i, 0)),
        ],
        out_shape=[
            jax.ShapeDtypeStruct((nblk * R, 128), jnp.float32),
            jax.ShapeDtypeStruct((nblk * R, 512), jnp.float32),
        ],
        scratch_shapes=[pltpu.VMEM((K * R, 128), jnp.float32)],
    )(g1, dnb, krt, w1, b1, u1w, u1b, sw, sb)


def _tc_b(g2, dnb, krt, w2, b2, u2w, u2b, bw, bb, sc, blk0):
    nblk = g2.shape[0] // RH
    return pl.pallas_call(
        _tc_b_body,
        grid=(nblk,),
        in_specs=[
            pl.BlockSpec((RH, 128), lambda i: (i, 0)),
            pl.BlockSpec((8, RH), lambda i: (0, i + blk0)),
            _const_spec((16, 8)),
            _const_spec((K, 128, 128)),
            _const_spec((1, 128)),
            _const_spec((128, 512)),
            _const_spec((1, 512)),
            _const_spec((512, 256)),
            _const_spec((1, 256)),
            pl.BlockSpec((R, 512), lambda i: (i, 0)),
        ],
        out_specs=pl.BlockSpec((R, 256), lambda i: (i, 0)),
        out_shape=jax.ShapeDtypeStruct((nblk * R, 256), jnp.float32),
        scratch_shapes=[pltpu.VMEM((K * R, 128), jnp.float32)],
    )(g2, dnb, krt, w2, b2, u2w, u2b, bw, bb, sc)


def _krt(kp):
    base = jnp.concatenate([kp, jnp.sum(kp * kp, 1, keepdims=True)], 1)
    return jnp.pad(base, ((0, 1), (0, 4)))


def kernel(features, points, neighbors, kp1, w1, b1, u1w, u1b, kp2, w2, b2,
           u2w, u2b, sw, sb, bw, bb):
    idx2d = jnp.pad(neighbors, ((0, NP - N), (0, 0))).reshape(M // 128, 128)
    px = jnp.pad(points[:, 0], (0, NP - N))
    py = jnp.pad(points[:, 1], (0, NP - N))
    pz = jnp.pad(points[:, 2], (0, NP - N))
    kr1 = _krt(kp1)
    kr2 = _krt(kp2)
    b1r = b1.reshape(1, -1)
    u1br = u1b.reshape(1, -1)
    sbr = sb.reshape(1, -1)
    b2r = b2.reshape(1, -1)
    u2br = u2b.reshape(1, -1)
    bbr = bb.reshape(1, -1)

    S = 5
    srow = M // S // 128
    sblk = NP // R // S
    idx_s = [idx2d[i * srow:(i + 1) * srow] for i in range(S)]

    dnb = _sc_dnb(px, py, pz, idx2d)
    g1 = [_sc_gather(features, ix) for ix in idx_s]
    ta = [_tc_a(g1[i], dnb, kr1, w1, b1r, u1w, u1br, sw, sbr, i * sblk)
          for i in range(S)]
    y = jnp.concatenate([t[0] for t in ta], 0)
    g2 = [_sc_gather(y, ix) for ix in idx_s]
    outs = [_tc_b(g2[i], dnb, kr2, w2, b2r, u2w, u2br, bw, bbr, ta[i][1],
                  i * sblk) for i in range(S)]
    out = jnp.concatenate(outs, 0)
    return out[:N].T[None]
